# baseline (speedup 1.0000x reference)
import functools

import jax
import jax.numpy as jnp
from jax import lax
from jax.experimental import pallas as pl
from jax.experimental.pallas import tpu as pltpu
from jax.experimental.pallas import tpu_sc as plsc

NUM_CORES = 2
NUM_SUBCORES = 16
NW = NUM_CORES * NUM_SUBCORES

CHUNK = 512
NBUF = 2


@functools.partial(jax.jit, static_argnames=("b_per_w", "n_groups"))
def _gather_rows(idx2, weight, *, b_per_w, n_groups):
    B = idx2.shape[0] * idx2.shape[1]
    D = weight.shape[1]
    mesh = plsc.VectorSubcoreMesh(core_axis_name="c", subcore_axis_name="s")

    @functools.partial(
        pl.kernel,
        mesh=mesh,
        out_type=jax.ShapeDtypeStruct((B, D), jnp.float32),
        scratch_types=[
            pltpu.VMEM((b_per_w // CHUNK, CHUNK), jnp.int32),
            pltpu.VMEM((NBUF, CHUNK, D), jnp.float32),
            pltpu.SemaphoreType.DMA,
            pltpu.SemaphoreType.DMA,
            pltpu.SemaphoreType.DMA,
            pltpu.SemaphoreType.DMA,
        ],
        compiler_params=pltpu.CompilerParams(use_tc_tiling_on_sc=False),
    )
    def k(idx_hbm, table_hbm, out_hbm, idx_v, rows_v, g0, g1, o0, o1):
        gsems = (g0, g1)
        osems = (o0, o1)
        wid = lax.axis_index("s") * NUM_CORES + lax.axis_index("c")
        base = wid * b_per_w
        n_chunks = b_per_w // CHUNK
        pltpu.sync_copy(idx_hbm.at[pl.ds(wid * n_chunks, n_chunks)], idx_v)

        def group(g, carry):
            descs = []
            for b in range(NBUF):
                j = g * NBUF + b
                off = base + j * CHUNK

                @pl.when(g > 0)
                def _drain():
                    pltpu.make_async_copy(
                        rows_v.at[b],
                        out_hbm.at[pl.ds(off - NBUF * CHUNK, CHUNK)],
                        osems[b],
                    ).wait()

                descs.append(
                    pltpu.async_copy(table_hbm.at[idx_v.at[j]], rows_v.at[b],
                                     gsems[b]))
            for b in range(NBUF):
                off = base + (g * NBUF + b) * CHUNK
                descs[b].wait()
                pltpu.async_copy(rows_v.at[b], out_hbm.at[pl.ds(off, CHUNK)],
                                 osems[b])
            return carry

        lax.fori_loop(0, n_groups, group, 0)
        for b in range(NBUF):
            off = base + ((n_groups - 1) * NBUF + b) * CHUNK
            pltpu.make_async_copy(
                rows_v.at[b], out_hbm.at[pl.ds(off, CHUNK)], osems[b]).wait()

    return k(idx2, weight)


def kernel(input_, weight):
    B = input_.shape[0] * input_.shape[1]
    idx2 = input_.reshape(B // CHUNK, CHUNK).astype(jnp.int32)
    b_per_w = B // NW
    n_groups = b_per_w // (CHUNK * NBUF)
    out = _gather_rows(idx2, weight, b_per_w=b_per_w, n_groups=n_groups)
    return out.reshape(input_.shape[0], input_.shape[1], weight.shape[1])


# speedup vs baseline: 1.0023x; 1.0023x over previous
import functools

import jax
import jax.numpy as jnp
from jax import lax
from jax.experimental import pallas as pl
from jax.experimental.pallas import tpu as pltpu
from jax.experimental.pallas import tpu_sc as plsc

NUM_CORES = 2
NUM_SUBCORES = 16
NW = NUM_CORES * NUM_SUBCORES

CHUNK = 320
NBUF = 4


@functools.partial(jax.jit, static_argnames=("b_per_w", "n_groups"))
def _gather_rows(idx2, weight, *, b_per_w, n_groups):
    B = idx2.shape[0] * idx2.shape[1]
    D = weight.shape[1]
    mesh = plsc.VectorSubcoreMesh(core_axis_name="c", subcore_axis_name="s")

    @functools.partial(
        pl.kernel,
        mesh=mesh,
        out_type=jax.ShapeDtypeStruct((B, D), jnp.float32),
        scratch_types=[
            pltpu.VMEM((b_per_w // CHUNK, CHUNK), jnp.int32),
            pltpu.VMEM((NBUF, CHUNK, D), jnp.float32),
            pltpu.SemaphoreType.DMA,
            pltpu.SemaphoreType.DMA,
            pltpu.SemaphoreType.DMA,
            pltpu.SemaphoreType.DMA,
            pltpu.SemaphoreType.DMA,
            pltpu.SemaphoreType.DMA,
            pltpu.SemaphoreType.DMA,
            pltpu.SemaphoreType.DMA,
        ],
        compiler_params=pltpu.CompilerParams(use_tc_tiling_on_sc=False),
    )
    def k(idx_hbm, table_hbm, out_hbm, idx_v, rows_v,
          g0, g1, g2, g3, o0, o1, o2, o3):
        gsems = (g0, g1, g2, g3)
        osems = (o0, o1, o2, o3)
        wid = lax.axis_index("s") * NUM_CORES + lax.axis_index("c")
        base = wid * b_per_w
        n_chunks = b_per_w // CHUNK
        pltpu.sync_copy(idx_hbm.at[pl.ds(wid * n_chunks, n_chunks)], idx_v)

        def group(g, carry):
            descs = []
            for b in range(NBUF):
                j = g * NBUF + b
                off = base + j * CHUNK

                @pl.when(g > 0)
                def _drain():
                    pltpu.make_async_copy(
                        rows_v.at[b],
                        out_hbm.at[pl.ds(off - NBUF * CHUNK, CHUNK)],
                        osems[b],
                    ).wait()

                descs.append(
                    pltpu.async_copy(table_hbm.at[idx_v.at[j]], rows_v.at[b],
                                     gsems[b]))
            for b in range(NBUF):
                off = base + (g * NBUF + b) * CHUNK
                descs[b].wait()
                pltpu.async_copy(rows_v.at[b], out_hbm.at[pl.ds(off, CHUNK)],
                                 osems[b])
            return carry

        lax.fori_loop(0, n_groups, group, 0)
        for b in range(NBUF):
            off = base + ((n_groups - 1) * NBUF + b) * CHUNK
            pltpu.make_async_copy(
                rows_v.at[b], out_hbm.at[pl.ds(off, CHUNK)], osems[b]).wait()

    return k(idx2, weight)


def kernel(input_, weight):
    B = input_.shape[0] * input_.shape[1]
    idx2 = input_.reshape(B // CHUNK, CHUNK).astype(jnp.int32)
    b_per_w = B // NW
    n_groups = b_per_w // (CHUNK * NBUF)
    out = _gather_rows(idx2, weight, b_per_w=b_per_w, n_groups=n_groups)
    return out.reshape(input_.shape[0], input_.shape[1], weight.shape[1])
